# baseline (speedup 1.0000x reference)
import functools

import jax
import jax.numpy as jnp
from jax import lax
from jax.experimental import pallas as pl
from jax.experimental.pallas import tpu as pltpu
from jax.experimental.pallas import tpu_sc as plsc

VOCAB = 8193
VPAD = 8208
RQ = 32
L = 8
D = 64

NC, NS, LANES = 2, 16, 16
NW = NC * NS

TB = 128
HT = TB // 2
SPM_SLOTS = 5
NSP = SPM_SLOTS * HT
NHB = (L - SPM_SLOTS) * HT


def _table_body(emb_ref, wr_ref, b_ref, out_ref):
    l = pl.program_id(0)
    t = jnp.dot(emb_ref[...], wr_ref[0], preferred_element_type=jnp.float32)
    t = t + b_ref[...] * (l == 0).astype(jnp.float32)
    out_ref[...] = t.astype(jnp.bfloat16)


def _build_table(emb_pad, wr, b2d):
    return pl.pallas_call(
        _table_body,
        grid=(L,),
        in_specs=[
            pl.BlockSpec((VPAD, RQ), lambda l: (0, 0)),
            pl.BlockSpec((1, RQ, D), lambda l: (l, 0, 0)),
            pl.BlockSpec((1, D), lambda l: (0, 0)),
        ],
        out_specs=pl.BlockSpec((VPAD, D), lambda l: (l, 0)),
        out_shape=jax.ShapeDtypeStruct((L * VPAD, D), jnp.bfloat16),
    )(emb_pad, wr, b2d)


def _make_sc_lookup(bsz, nsz):
    bt_n = bsz // TB
    nblk = nsz * bt_n
    nb = nblk // NW
    mesh = plsc.VectorSubcoreMesh(core_axis_name="c", subcore_axis_name="s")

    @functools.partial(
        pl.kernel,
        out_type=jax.ShapeDtypeStruct((nsz * D * bsz,), jnp.float32),
        mesh=mesh,
        compiler_params=pltpu.CompilerParams(use_tc_tiling_on_sc=False,
                                             needs_layout_passes=False),
        scratch_types=[
            pltpu.VMEM((2, TB * L), jnp.int32),
            pltpu.VMEM((2, 2 * L, HT), jnp.int32),
            pltpu.VMEM((2, HT * L, D), jnp.bfloat16),
            pltpu.VMEM((D * TB,), jnp.float32),
            pltpu.VMEM_SHARED((SPM_SLOTS * VPAD, D), jnp.bfloat16),
            pltpu.SemaphoreType.DMA,
            pltpu.SemaphoreType.DMA,
            pltpu.SemaphoreType.DMA,
            pltpu.SemaphoreType.DMA,
            pltpu.SemaphoreType.DMA,
        ],
    )
    def sc_lookup(table_hbm, codes_hbm, out_hbm, codes_v, idx_v, rows_v,
                  stage_v, shared, semc0, semc1, semg0, semg1, semo):
        wid = lax.axis_index("s") * NC + lax.axis_index("c")
        blk0 = wid * nb
        semc = (semc0, semc1)
        semg = (semg0, semg1)
        base = lax.iota(jnp.int32, LANES) * (2 * TB)
        bases = [base + (h * 32 * TB + p * TB)
                 for h in range(D // 32) for p in range(2)]

        def codes_copy(b, cbuf):
            fb = blk0 + b
            return pltpu.make_async_copy(
                codes_hbm.at[pl.ds(fb * (TB * L), TB * L)],
                codes_v.at[cbuf], semc[cbuf])

        def gather_copies(cbuf, h, rbuf):
            cps = []
            for lr in range(L):
                src = table_hbm
                cps.append(pltpu.make_async_copy(
                    src.at[idx_v.at[cbuf, h * L + lr]],
                    rows_v.at[rbuf, pl.ds(lr * HT, HT)],
                    semg[rbuf]))
            return cps

        def out_copies(b):
            fb = blk0 + b
            n = fb // bt_n
            bt = fb % bt_n
            return [
                pltpu.make_async_copy(
                    stage_v.at[pl.ds(dt * (8 * TB), 8 * TB)],
                    out_hbm.at[pl.ds(((n * L + dt) * bt_n + bt) * (8 * TB),
                                     8 * TB)],
                    semo)
                for dt in range(D // 8)
            ]

        def compute_idx(cbuf):
            for i in range(TB * L // LANES):
                lr = i // 8
                col = (i % 8) * LANES
                h, brh = col // HT, col % HT
                v = codes_v[cbuf, pl.ds(i * LANES, LANES)] + lr * VPAD
                idx_v[cbuf, h * L + lr, pl.ds(brh, LANES)] = v

        def accum_half(h, rbuf):
            @plsc.parallel_loop(0, HT, unroll=8)
            def tok_body(tt):
                t = h * HT + tt
                for h32 in range(D // 32):
                    r = [rows_v[rbuf, s * HT + tt, pl.ds(h32 * 32, 32)]
                         for s in range(L)]
                    while len(r) > 1:
                        r = [r[i] + r[i + 1] for i in range(0, len(r), 2)]
                    even, odd = plsc.unpack(r[0],
                                            format=plsc.PackFormat.INTERLEAVED)
                    plsc.store_scatter(stage_v, [bases[2 * h32] + t], even)
                    plsc.store_scatter(stage_v, [bases[2 * h32 + 1] + t], odd)

        def phase(b, cbufx, cbufy):
            bp1 = jnp.minimum(b + 1, nb - 1)
            bp2 = jnp.minimum(b + 2, nb - 1)
            for cp in gather_copies(cbufx, 1, 1):
                cp.start()
            codes_copy(bp1, cbufy).wait()
            compute_idx(cbufy)
            codes_copy(bp2, cbufx).start()
            for cp in gather_copies(cbufx, 0, 0):
                cp.wait()
            for cp in out_copies(jnp.maximum(b - 1, 0)):
                cp.wait()
            accum_half(0, 0)
            for cp in gather_copies(cbufy, 0, 0):
                cp.start()
            for cp in gather_copies(cbufx, 1, 1):
                cp.wait()
            accum_half(1, 1)
            for cp in out_copies(b):
                cp.start()

        sid = lax.axis_index("s")
        spm_rows = SPM_SLOTS * VPAD // NS
        pltpu.sync_copy(table_hbm.at[pl.ds(sid * spm_rows, spm_rows)],
                        shared.at[pl.ds(sid * spm_rows, spm_rows)])
        plsc.subcore_barrier()

        codes_copy(0, 0).start()
        for cp in out_copies(0):
            cp.start()
        codes_copy(0, 0).wait()
        compute_idx(0)
        for cp in gather_copies(0, 0, 0):
            cp.start()
        codes_copy(1, 1).start()

        def pair_body(p, carry):
            phase(2 * p, 0, 1)
            phase(2 * p + 1, 1, 0)
            return carry

        lax.fori_loop(0, nb // 2, pair_body, 0)

        for cp in gather_copies(1, 0, 0):
            cp.wait()
        codes_copy(nb - 1, 1).wait()
        for cp in out_copies(nb - 1):
            cp.wait()

    return sc_lookup


def kernel(codes, emb_table, W, b):
    bsz, nsz, lsz = codes.shape
    wr = W.reshape(D, L, RQ).transpose(1, 2, 0)
    emb_pad = jnp.pad(emb_table, ((0, VPAD - VOCAB), (0, 0)))
    table = _build_table(emb_pad, wr, b.reshape(1, D))
    codes_t = (codes.transpose(1, 2, 0)
               .reshape(nsz, lsz, bsz // TB, TB)
               .transpose(0, 2, 1, 3)
               .reshape(-1))
    out = _make_sc_lookup(bsz, nsz)(table, codes_t)
    out = (out.reshape(nsz, D // 8, bsz // TB, 8, TB)
           .transpose(2, 4, 0, 1, 3)
           .reshape(bsz, nsz, D))
    return out


# speedup vs baseline: 1.0734x; 1.0734x over previous
import functools

import jax
import jax.numpy as jnp
from jax import lax
from jax.experimental import pallas as pl
from jax.experimental.pallas import tpu as pltpu
from jax.experimental.pallas import tpu_sc as plsc

VOCAB = 8193
VPAD = 8208
RQ = 32
L = 8
D = 64

NC, NS, LANES = 2, 16, 16
NW = NC * NS

TB = 128


def _table_body(emb_ref, wr_ref, b_ref, out_ref):
    l = pl.program_id(0)
    t = jnp.dot(emb_ref[...], wr_ref[0], preferred_element_type=jnp.float32)
    t = t + b_ref[...] * (l == 0).astype(jnp.float32)
    out_ref[...] = t.astype(jnp.bfloat16)


def _build_table(emb_pad, wr, b2d):
    return pl.pallas_call(
        _table_body,
        grid=(L,),
        in_specs=[
            pl.BlockSpec((VPAD, RQ), lambda l: (0, 0)),
            pl.BlockSpec((1, RQ, D), lambda l: (l, 0, 0)),
            pl.BlockSpec((1, D), lambda l: (0, 0)),
        ],
        out_specs=pl.BlockSpec((VPAD, D), lambda l: (l, 0)),
        out_shape=jax.ShapeDtypeStruct((L * VPAD, D), jnp.bfloat16),
    )(emb_pad, wr, b2d)


def _make_sc_lookup(bsz, nsz):
    bt_n = bsz // TB
    nblk = nsz * bt_n
    nb = nblk // NW
    mesh = plsc.VectorSubcoreMesh(core_axis_name="c", subcore_axis_name="s")

    @functools.partial(
        pl.kernel,
        out_type=jax.ShapeDtypeStruct((nsz * D * bsz,), jnp.float32),
        mesh=mesh,
        compiler_params=pltpu.CompilerParams(use_tc_tiling_on_sc=False,
                                             needs_layout_passes=False),
        scratch_types=[
            pltpu.VMEM((2, TB * L), jnp.int32),
            pltpu.VMEM((2, TB * L), jnp.int32),
            pltpu.VMEM((2, TB * L, D), jnp.bfloat16),
            pltpu.VMEM((2, D * TB), jnp.float32),
            pltpu.SemaphoreType.DMA,
            pltpu.SemaphoreType.DMA,
            pltpu.SemaphoreType.DMA,
            pltpu.SemaphoreType.DMA,
            pltpu.SemaphoreType.DMA,
            pltpu.SemaphoreType.DMA,
        ],
    )
    def sc_lookup(table_hbm, codes_hbm, out_hbm, codes_v, idx_v, rows_v,
                  stage_v, semc0, semc1, semg0, semg1, semo0, semo1):
        wid = lax.axis_index("s") * NC + lax.axis_index("c")
        blk0 = wid * nb
        semc = (semc0, semc1)
        semg = (semg0, semg1)
        semo = (semo0, semo1)
        base = lax.iota(jnp.int32, LANES) * (2 * TB)
        bases = [base + (h * 32 * TB + p * TB)
                 for h in range(D // 32) for p in range(2)]

        def codes_copy(b, buf):
            fb = blk0 + b
            return pltpu.make_async_copy(
                codes_hbm.at[pl.ds(fb * (TB * L), TB * L)],
                codes_v.at[buf], semc[buf])

        def gather_copies(buf):
            return [
                pltpu.make_async_copy(
                    table_hbm.at[idx_v.at[buf]],
                    rows_v.at[buf],
                    semg[buf])
            ]

        def out_copies(b, buf):
            fb = blk0 + b
            n = fb // bt_n
            bt = fb % bt_n
            return [
                pltpu.make_async_copy(
                    stage_v.at[buf, pl.ds(dt * (8 * TB), 8 * TB)],
                    out_hbm.at[pl.ds(((n * L + dt) * bt_n + bt) * (8 * TB),
                                     8 * TB)],
                    semo[buf])
                for dt in range(D // 8)
            ]

        def compute_idx(buf):
            for i in range(TB * L // LANES):
                v = codes_v[buf, pl.ds(i * LANES, LANES)] + (i // 8) * VPAD
                idx_v[buf, pl.ds(i * LANES, LANES)] = v

        def accum_store(b, buf):
            for cp in out_copies(b, buf):
                cp.wait()

            @plsc.parallel_loop(0, TB, unroll=8)
            def tok_body(t):
                for h in range(D // 32):
                    r = [rows_v[buf, s * 128 + t, pl.ds(h * 32, 32)]
                         for s in range(L)]
                    while len(r) > 1:
                        r = [r[i] + r[i + 1] for i in range(0, len(r), 2)]
                    a = r[0]
                    even, odd = plsc.unpack(a, format=plsc.PackFormat.INTERLEAVED)
                    plsc.store_scatter(stage_v.at[buf], [bases[2 * h] + t], even)
                    plsc.store_scatter(stage_v.at[buf], [bases[2 * h + 1] + t], odd)
            for cp in out_copies(b, buf):
                cp.start()

        def phase(b, bufx, bufy):
            bp1 = jnp.minimum(b + 1, nb - 1)
            bp2 = jnp.minimum(b + 2, nb - 1)
            codes_copy(bp1, bufy).wait()
            compute_idx(bufy)
            for cp in gather_copies(bufy):
                cp.start()
            codes_copy(bp2, bufx).start()
            for cp in gather_copies(bufx):
                cp.wait()
            accum_store(b, bufx)

        codes_copy(0, 0).start()
        for buf in (0, 1):
            for cp in out_copies(buf, buf):
                cp.start()
        codes_copy(0, 0).wait()
        compute_idx(0)
        for cp in gather_copies(0):
            cp.start()
        codes_copy(1, 1).start()

        def pair_body(p, carry):
            phase(2 * p, 0, 1)
            phase(2 * p + 1, 1, 0)
            return carry

        lax.fori_loop(0, nb // 2, pair_body, 0)

        for cp in gather_copies(0):
            cp.wait()
        codes_copy(nb - 1, 1).wait()
        for buf in (0, 1):
            for cp in out_copies(nb - 2 + buf, buf):
                cp.wait()

    return sc_lookup


def kernel(codes, emb_table, W, b):
    bsz, nsz, lsz = codes.shape
    wr = W.reshape(D, L, RQ).transpose(1, 2, 0)
    emb_pad = jnp.pad(emb_table, ((0, VPAD - VOCAB), (0, 0)))
    table = _build_table(emb_pad, wr, b.reshape(1, D))
    codes_t = (codes.transpose(1, 2, 0)
               .reshape(nsz, lsz, bsz // TB, TB)
               .transpose(0, 2, 1, 3)
               .reshape(-1))
    out = _make_sc_lookup(bsz, nsz)(table, codes_t)
    out = (out.reshape(nsz, D // 8, bsz // TB, 8, TB)
           .transpose(2, 4, 0, 1, 3)
           .reshape(bsz, nsz, D))
    return out
